# baseline (speedup 1.0000x reference)
import functools
import numpy as np
import jax
import jax.numpy as jnp
from jax import lax
from jax.experimental import pallas as pl
from jax.experimental.pallas import tpu as pltpu
from jax.experimental.pallas import tpu_sc as plsc

N = 10000
E = 320000
BS = 16
D_H = 128
D_E = 16
D_T = 32
DIMS = 3
H4 = 4 * D_H
H2 = 2 * D_H

NW = 32
CH = 80
TPS = N // 16

E_SPLITS = (163840, 156160)
BLKN = 2000
BLKE = 2560

def _silu(x):
    return x * jax.nn.sigmoid(x)


def _tfe_body(t_ref, batch_ref, frame_ref, tw1, tb1, tw2, tb2, out_ref):
    half = D_T // 2
    freqs = jnp.exp(
        (-np.log(10000.0) / half)
        * lax.broadcasted_iota(jnp.int32, (1, half), 1).astype(jnp.float32))
    args = t_ref[...] * freqs
    temb = jnp.concatenate([jnp.cos(args), jnp.sin(args)], axis=-1)
    th = _silu(jnp.dot(temb, tw1[...], preferred_element_type=jnp.float32)
               + tb1[...])
    th = jnp.dot(th, tw2[...], preferred_element_type=jnp.float32) + tb2[...]
    oh = (batch_ref[...] == lax.broadcasted_iota(jnp.int32, (1, BS), 1))
    ten = jnp.dot(oh.astype(jnp.float32), th,
                  preferred_element_type=jnp.float32)
    out_ref[...] = jnp.concatenate([ten, frame_ref[...]],
                                   axis=-1).astype(out_ref.dtype)


@functools.lru_cache(maxsize=None)
def _sc_mesh():
    return plsc.VectorSubcoreMesh(core_axis_name="c", subcore_axis_name="s")


def _make_sc_gather(tables_meta, n_e):
    n_t = len(tables_meta)
    ew = n_e // NW
    nch = ew // CH
    out_type = tuple(jax.ShapeDtypeStruct((n_e, w), d)
                     for _, w, d in tables_meta)
    scratch = [pltpu.VMEM((CH,), jnp.int32), pltpu.VMEM((CH,), jnp.int32)]
    scratch += [pltpu.VMEM((CH, w), d) for _, w, d in tables_meta]
    scratch += [pltpu.SemaphoreType.DMA]

    @functools.partial(pl.kernel, mesh=_sc_mesh(), out_type=out_type,
                       scratch_types=scratch,
                       compiler_params=pltpu.CompilerParams(
                           use_tc_tiling_on_sc=False))
    def gather_kernel(src_hbm, dst_hbm, *rest):
        tabs = rest[:n_t]
        outs = rest[n_t:2 * n_t]
        si = rest[2 * n_t]
        di = rest[2 * n_t + 1]
        bufs = rest[2 * n_t + 2: 2 * n_t + 2 + n_t]
        sem = rest[-1]
        wid = lax.axis_index("s") * 2 + lax.axis_index("c")
        base = wid * ew

        def body(j, carry):
            off = pl.multiple_of(base + j * CH, 8)
            pltpu.sync_copy(src_hbm.at[pl.ds(off, CH)], si)
            pltpu.sync_copy(dst_hbm.at[pl.ds(off, CH)], di)
            copies = []
            for (use_dst, _, _), tab, buf in zip(tables_meta, tabs, bufs):
                idx = di if use_dst else si
                copies.append(pltpu.async_copy(tab.at[idx], buf, sem))
            for c in copies:
                c.wait()
            for buf, out in zip(bufs, outs):
                pltpu.sync_copy(buf, out.at[pl.ds(off, CH)])
            return carry

        lax.fori_loop(0, nch, body, 0)

    return gather_kernel


@functools.lru_cache(maxsize=None)
def _make_sc_scatter_add(n_e):
    ew = n_e // NW
    nch = ew // CH

    @functools.partial(
        pl.kernel, mesh=_sc_mesh(),
        out_type=jax.ShapeDtypeStruct((2 * N, 32), jnp.float32),
        scratch_types=[pltpu.VMEM((CH,), jnp.int32),
                       pltpu.VMEM((CH, 32), jnp.float32),
                       pltpu.VMEM_SHARED((N, 32), jnp.float32)],
        compiler_params=pltpu.CompilerParams(use_tc_tiling_on_sc=False))
    def scatter_kernel(dst_hbm, comb_hbm, zeros_hbm, out_hbm, di, buf, acc):
        c = lax.axis_index("c")
        s = lax.axis_index("s")
        wid = s * 2 + c
        r0 = s * TPS
        pltpu.sync_copy(zeros_hbm.at[pl.ds(r0, TPS)], acc.at[pl.ds(r0, TPS)])
        plsc.subcore_barrier()
        base = wid * ew

        def body(j, carry):
            off = pl.multiple_of(base + j * CH, 8)
            pltpu.sync_copy(dst_hbm.at[pl.ds(off, CH)], di)
            pltpu.sync_copy(comb_hbm.at[pl.ds(off, CH)], buf)
            pltpu.sync_copy(buf, acc.at[di], add=True)
            return carry

        lax.fori_loop(0, nch, body, 0)
        plsc.subcore_barrier()
        pltpu.sync_copy(acc.at[pl.ds(r0, TPS)],
                        out_hbm.at[pl.ds(c * N + r0, TPS)])

    return scatter_kernel


def _msg_body(sx, dx, ea, di, tfe, ps, pd,
              w_sx, w_dx, w_ea, w_di, w_tfe, b1, w2, b2,
              pw1, pb1, pw2, pb2, out_ref):
    h = jnp.dot(sx[...], w_sx[...], preferred_element_type=jnp.float32)
    h = h + jnp.dot(dx[...], w_dx[...], preferred_element_type=jnp.float32)
    h = h + jnp.dot(ea[...], w_ea[...], preferred_element_type=jnp.float32)
    h = h + jnp.dot(tfe[...], w_tfe[...], preferred_element_type=jnp.float32)
    h = h + di[...] * w_di[...]
    h = _silu(h + b1[...])
    msg = _silu(jnp.dot(h, w2[...], preferred_element_type=jnp.float32)
                + b2[...])
    ph = _silu(jnp.dot(msg, pw1[...], preferred_element_type=jnp.float32)
               + pb1[...])
    praw = jnp.dot(ph, pw2[...], preferred_element_type=jnp.float32) + pb2[...]
    lane = lax.broadcasted_iota(jnp.int32, (1, D_E), 1)
    cnt_lane = (lane == DIMS).astype(jnp.float32)
    ppart = praw * (pd[...] - ps[...]) + cnt_lane
    out_ref[...] = jnp.concatenate([msg, ppart], axis=-1)


def _node_body(x, p0, p1, p2, p3, te, w_x, w_a, w_t, b1, w2, b2, g, bb,
               xn_ref, pv_ref):
    psum = (p0[...] + p1[...]) + (p2[...] + p3[...])
    aggr = psum[:, :D_E]
    pos3 = psum[:, D_E:D_E + DIMS]
    cnt = psum[:, D_E + DIMS:D_E + DIMS + 1]
    pv_ref[...] = pos3 / jnp.maximum(cnt, 1.0)
    xv = x[...]
    h = jnp.dot(xv, w_x[...], preferred_element_type=jnp.float32)
    h = h + jnp.dot(aggr, w_a[...], preferred_element_type=jnp.float32)
    h = h + jnp.dot(te[...][:, :D_T].astype(jnp.float32), w_t[...],
                    preferred_element_type=jnp.float32)
    h = _silu(h + b1[...])
    h = jnp.dot(h, w2[...], preferred_element_type=jnp.float32) + b2[...] + xv
    m = jnp.mean(h, axis=-1, keepdims=True)
    v = jnp.mean((h - m) ** 2, axis=-1, keepdims=True)
    xn_ref[...] = (h - m) * lax.rsqrt(v + 1e-5) * g[...] + bb[...]


def _edge_body(xs, xd, di, te, ea, w_xs, w_xd, w_di, w_te, b1, w2, b2, g, bb,
               out_ref):
    h = jnp.dot(xs[...], w_xs[...], preferred_element_type=jnp.float32)
    h = h + jnp.dot(xd[...], w_xd[...], preferred_element_type=jnp.float32)
    h = h + jnp.dot(te[...][:, :D_T], w_te[...],
                    preferred_element_type=jnp.float32)
    h = h + di[...] * w_di[...]
    h = _silu(h + b1[...])
    h = jnp.dot(h, w2[...], preferred_element_type=jnp.float32) + b2[...]
    h = h + ea[...]
    m = jnp.mean(h, axis=-1, keepdims=True)
    v = jnp.mean((h - m) ** 2, axis=-1, keepdims=True)
    out_ref[...] = (h - m) * lax.rsqrt(v + 1e-5) * g[...] + bb[...]


_META_B = ((False, D_H, jnp.bfloat16), (True, D_H, jnp.bfloat16),
           (False, 2 * D_T, jnp.bfloat16), (False, D_E, jnp.float32),
           (True, D_E, jnp.float32))
_META_F = ((False, D_H, jnp.bfloat16), (True, D_H, jnp.bfloat16))


@functools.lru_cache(maxsize=None)
def _gather_b(n_e):
    return _make_sc_gather(_META_B, n_e)


@functools.lru_cache(maxsize=None)
def _gather_f(n_e):
    return _make_sc_gather(_META_F, n_e)


def _row(v):
    return v.reshape(1, -1)


def kernel(x_feat, t, pos, edge_index, edge_attr, dist, batch, frame_emb,
           params):
    p = params
    src = edge_index[0].astype(jnp.int32)
    dst = edge_index[1].astype(jnp.int32)
    t2 = t.reshape(BS, 1)
    batch2 = batch.astype(jnp.int32).reshape(N, 1)
    posp = jnp.pad(pos, ((0, 0), (0, D_E - DIMS)))

    tfe = pl.pallas_call(
        _tfe_body,
        grid=(N // BLKN,),
        in_specs=[
            pl.BlockSpec((BS, 1), lambda i: (0, 0)),
            pl.BlockSpec((BLKN, 1), lambda i: (i, 0)),
            pl.BlockSpec((BLKN, D_T), lambda i: (i, 0)),
            pl.BlockSpec((D_T, H2), lambda i: (0, 0)),
            pl.BlockSpec((1, H2), lambda i: (0, 0)),
            pl.BlockSpec((H2, D_T), lambda i: (0, 0)),
            pl.BlockSpec((1, D_T), lambda i: (0, 0)),
        ],
        out_specs=pl.BlockSpec((BLKN, 2 * D_T), lambda i: (i, 0)),
        out_shape=jax.ShapeDtypeStruct((N, 2 * D_T), jnp.float32),
    )(t2, batch2, frame_emb, p['time_w1'], _row(p['time_b1']), p['time_w2'],
      _row(p['time_b2']))

    e0 = E_SPLITS[0]
    srcs = (src[:e0], src[e0:])
    dsts = (dst[:e0], dst[e0:])
    eas = (edge_attr[:e0], edge_attr[e0:])
    dis = (dist[:e0], dist[e0:])

    mw1 = p['msg_w1']
    pw2p = jnp.pad(p['pos_w2'], ((0, 0), (0, D_E - DIMS)))
    pb2p = _row(jnp.pad(p['pos_b2'], (0, D_E - DIMS)))
    wspec = lambda a: pl.BlockSpec(a.shape, lambda i: (0,) * a.ndim)
    bf = jnp.bfloat16
    msg_ws = [mw1[:D_H].astype(bf), mw1[D_H:2 * D_H].astype(bf),
              mw1[2 * D_H:2 * D_H + D_E],
              mw1[2 * D_H + D_E:2 * D_H + D_E + 1],
              mw1[2 * D_H + D_E + 1:].astype(bf),
              _row(p['msg_b1']), p['msg_w2'], _row(p['msg_b2']),
              p['pos_w1'], _row(p['pos_b1']), pw2p, pb2p]
    ew1 = p['edge_w1']
    edge_ws = [ew1[:D_H].astype(bf), ew1[D_H:2 * D_H].astype(bf),
               ew1[2 * D_H:2 * D_H + 1], ew1[2 * D_H + 1:].astype(bf),
               _row(p['edge_b1']), p['edge_w2'],
               _row(p['edge_b2']), _row(p['ln_e_g']), _row(p['ln_e_b'])]
    zeros32 = jnp.zeros((N, 2 * D_E), jnp.float32)

    x_bf = x_feat.astype(bf)
    tfe_bf = tfe.astype(bf)
    parts = []
    tfe_es = []
    for k in range(2):
        n_e = E_SPLITS[k]
        sx, dx, tfe_e, ps, pd = _gather_b(n_e)(srcs[k], dsts[k], x_bf,
                                               x_bf, tfe_bf, posp, posp)
        tfe_es.append(tfe_e)
        espec = lambda w: pl.BlockSpec((BLKE, w), lambda i: (i, 0))
        comb = pl.pallas_call(
            _msg_body,
            grid=(n_e // BLKE,),
            in_specs=[espec(D_H), espec(D_H), espec(D_E), espec(1),
                      espec(2 * D_T), espec(D_E), espec(D_E)]
                     + [wspec(w) for w in msg_ws],
            out_specs=pl.BlockSpec((BLKE, 2 * D_E), lambda i: (i, 0)),
            out_shape=jax.ShapeDtypeStruct((n_e, 2 * D_E), jnp.float32),
        )(sx, dx, eas[k], dis[k], tfe_e, ps, pd, *msg_ws)
        part = _make_sc_scatter_add(n_e)(dsts[k], comb, zeros32)
        parts += [part[:N], part[N:]]

    nw1 = p['node_w1']
    node_ws = [nw1[:D_H], nw1[D_H:D_H + D_E], nw1[D_H + D_E:],
               _row(p['node_b1']), p['node_w2'], _row(p['node_b2']),
               _row(p['ln_n_g']), _row(p['ln_n_b'])]
    nspec = lambda w: pl.BlockSpec((BLKN, w), lambda i: (i, 0))
    x_new, pos_v_t = pl.pallas_call(
        _node_body,
        grid=(N // BLKN,),
        in_specs=[nspec(D_H)] + [nspec(2 * D_E)] * 4
                 + [pl.BlockSpec((BLKN, 2 * D_T), lambda i: (i, 0))]
                 + [wspec(w) for w in node_ws],
        out_specs=[nspec(D_H), nspec(DIMS)],
        out_shape=[jax.ShapeDtypeStruct((N, D_H), jnp.float32),
                   jax.ShapeDtypeStruct((N, DIMS), jnp.float32)],
    )(x_feat, *parts, tfe, *node_ws)

    edge_news = []
    xn_bf = x_new.astype(bf)
    for k in range(2):
        n_e = E_SPLITS[k]
        xs, xd = _gather_f(n_e)(srcs[k], dsts[k], xn_bf, xn_bf)
        espec = lambda w: pl.BlockSpec((BLKE, w), lambda i: (i, 0))
        edge_news.append(pl.pallas_call(
            _edge_body,
            grid=(n_e // BLKE,),
            in_specs=[espec(D_H), espec(D_H), espec(1),
                      pl.BlockSpec((BLKE, 2 * D_T), lambda i: (i, 0)),
                      espec(D_E)]
                     + [wspec(w) for w in edge_ws],
            out_specs=pl.BlockSpec((BLKE, D_E), lambda i: (i, 0)),
            out_shape=jax.ShapeDtypeStruct((n_e, D_E), jnp.float32),
        )(xs, xd, dis[k], tfe_es[k], eas[k], *edge_ws))
    edge_new = jnp.concatenate(edge_news, axis=0)

    return (pos_v_t, x_new, edge_new)


# speedup vs baseline: 1.4839x; 1.4839x over previous
import functools
import numpy as np
import jax
import jax.numpy as jnp
from jax import lax
from jax.experimental import pallas as pl
from jax.experimental.pallas import tpu as pltpu
from jax.experimental.pallas import tpu_sc as plsc

N = 10000
E = 320000
BS = 16
D_H = 128
D_E = 16
D_T = 32
DIMS = 3
H4 = 4 * D_H
H2 = 2 * D_H

NW = 32
CH = 80
TPS = N // 16
D_P = 8
CW = 24

E_SPLITS = (163840, 156160)
BLKN = 2000
BLKE = 2560

def _silu(x):
    return x * jax.nn.sigmoid(x)


def _tfe_body(t_ref, batch_ref, frame_ref, tw1, tb1, tw2, tb2, out_ref):
    half = D_T // 2
    freqs = jnp.exp(
        (-np.log(10000.0) / half)
        * lax.broadcasted_iota(jnp.int32, (1, half), 1).astype(jnp.float32))
    args = t_ref[...] * freqs
    temb = jnp.concatenate([jnp.cos(args), jnp.sin(args)], axis=-1)
    th = _silu(jnp.dot(temb, tw1[...], preferred_element_type=jnp.float32)
               + tb1[...])
    th = jnp.dot(th, tw2[...], preferred_element_type=jnp.float32) + tb2[...]
    oh = (batch_ref[...] == lax.broadcasted_iota(jnp.int32, (1, BS), 1))
    ten = jnp.dot(oh.astype(jnp.float32), th,
                  preferred_element_type=jnp.float32)
    out_ref[...] = jnp.concatenate([ten, frame_ref[...]],
                                   axis=-1).astype(out_ref.dtype)


@functools.lru_cache(maxsize=None)
def _sc_mesh():
    return plsc.VectorSubcoreMesh(core_axis_name="c", subcore_axis_name="s")


def _make_sc_gather(tables_meta, n_e):
    n_t = len(tables_meta)
    ew = n_e // NW
    nch = ew // CH
    out_type = tuple(jax.ShapeDtypeStruct((n_e, w), d)
                     for _, w, d in tables_meta)
    scratch = [pltpu.VMEM((CH,), jnp.int32), pltpu.VMEM((CH,), jnp.int32)]
    scratch += [pltpu.VMEM((CH, w), d) for _, w, d in tables_meta]
    scratch += [pltpu.SemaphoreType.DMA]

    @functools.partial(pl.kernel, mesh=_sc_mesh(), out_type=out_type,
                       scratch_types=scratch,
                       compiler_params=pltpu.CompilerParams(
                           use_tc_tiling_on_sc=False))
    def gather_kernel(src_hbm, dst_hbm, *rest):
        tabs = rest[:n_t]
        outs = rest[n_t:2 * n_t]
        si = rest[2 * n_t]
        di = rest[2 * n_t + 1]
        bufs = rest[2 * n_t + 2: 2 * n_t + 2 + n_t]
        sem = rest[-1]
        wid = lax.axis_index("s") * 2 + lax.axis_index("c")
        base = wid * ew

        def body(j, carry):
            off = pl.multiple_of(base + j * CH, 8)
            pltpu.sync_copy(src_hbm.at[pl.ds(off, CH)], si)
            pltpu.sync_copy(dst_hbm.at[pl.ds(off, CH)], di)
            copies = []
            for (use_dst, _, _), tab, buf in zip(tables_meta, tabs, bufs):
                idx = di if use_dst else si
                copies.append(pltpu.async_copy(tab.at[idx], buf, sem))
            for c in copies:
                c.wait()
            for buf, out in zip(bufs, outs):
                pltpu.sync_copy(buf, out.at[pl.ds(off, CH)])
            return carry

        lax.fori_loop(0, nch, body, 0)

    return gather_kernel


@functools.lru_cache(maxsize=None)
def _make_sc_scatter_add(n_e):
    ew = n_e // NW
    nch = ew // CH

    @functools.partial(
        pl.kernel, mesh=_sc_mesh(),
        out_type=jax.ShapeDtypeStruct((2 * N, CW), jnp.float32),
        scratch_types=[pltpu.VMEM((CH,), jnp.int32),
                       pltpu.VMEM((CH, CW), jnp.float32),
                       pltpu.VMEM_SHARED((N, CW), jnp.float32)],
        compiler_params=pltpu.CompilerParams(use_tc_tiling_on_sc=False))
    def scatter_kernel(dst_hbm, comb_hbm, zeros_hbm, out_hbm, di, buf, acc):
        c = lax.axis_index("c")
        s = lax.axis_index("s")
        wid = s * 2 + c
        r0 = s * TPS
        pltpu.sync_copy(zeros_hbm.at[pl.ds(r0, TPS)], acc.at[pl.ds(r0, TPS)])
        plsc.subcore_barrier()
        base = wid * ew

        def body(j, carry):
            off = pl.multiple_of(base + j * CH, 8)
            pltpu.sync_copy(dst_hbm.at[pl.ds(off, CH)], di)
            pltpu.sync_copy(comb_hbm.at[pl.ds(off, CH)], buf)
            pltpu.sync_copy(buf, acc.at[di], add=True)
            return carry

        lax.fori_loop(0, nch, body, 0)
        plsc.subcore_barrier()
        pltpu.sync_copy(acc.at[pl.ds(r0, TPS)],
                        out_hbm.at[pl.ds(c * N + r0, TPS)])

    return scatter_kernel


def _msg_body(sx, dx, ea, di, tfe, ps, pd,
              w_sx, w_dx, w_ea, w_di, w_tfe, b1, w2, b2,
              pw1, pb1, pw2, pb2, out_ref):
    bf = jnp.bfloat16
    h = jnp.dot(sx[...].astype(bf), w_sx[...],
                preferred_element_type=jnp.float32)
    h = h + jnp.dot(dx[...].astype(bf), w_dx[...],
                    preferred_element_type=jnp.float32)
    h = h + jnp.dot(ea[...], w_ea[...], preferred_element_type=jnp.float32)
    h = h + jnp.dot(tfe[...].astype(bf), w_tfe[...],
                    preferred_element_type=jnp.float32)
    h = h + di[...] * w_di[...]
    h = _silu(h + b1[...])
    msg = _silu(jnp.dot(h, w2[...], preferred_element_type=jnp.float32)
                + b2[...])
    ph = _silu(jnp.dot(msg, pw1[...], preferred_element_type=jnp.float32)
               + pb1[...])
    praw = jnp.dot(ph, pw2[...], preferred_element_type=jnp.float32) + pb2[...]
    lane = lax.broadcasted_iota(jnp.int32, (1, D_P), 1)
    cnt_lane = (lane == DIMS).astype(jnp.float32)
    ppart = praw * (pd[...] - ps[...]) + cnt_lane
    out_ref[...] = jnp.concatenate([msg, ppart], axis=-1)


def _node_body(x, p0, p1, p2, p3, te, w_x, w_a, w_t, b1, w2, b2, g, bb,
               xn_ref, pv_ref):
    psum = (p0[...] + p1[...]) + (p2[...] + p3[...])
    aggr = psum[:, :D_E]
    pos3 = psum[:, D_E:D_E + DIMS]
    cnt = psum[:, D_E + DIMS:D_E + DIMS + 1]
    pv_ref[...] = pos3 / jnp.maximum(cnt, 1.0)
    xv = x[...]
    h = jnp.dot(xv, w_x[...], preferred_element_type=jnp.float32)
    h = h + jnp.dot(aggr, w_a[...], preferred_element_type=jnp.float32)
    h = h + jnp.dot(te[...][:, :D_T].astype(jnp.float32), w_t[...],
                    preferred_element_type=jnp.float32)
    h = _silu(h + b1[...])
    h = jnp.dot(h, w2[...], preferred_element_type=jnp.float32) + b2[...] + xv
    m = jnp.mean(h, axis=-1, keepdims=True)
    v = jnp.mean((h - m) ** 2, axis=-1, keepdims=True)
    xn_ref[...] = (h - m) * lax.rsqrt(v + 1e-5) * g[...] + bb[...]


def _edge_body(xs, xd, di, te, ea, w_xs, w_xd, w_di, w_te, b1, w2, b2, g, bb,
               out_ref):
    bf = jnp.bfloat16
    h = jnp.dot(xs[...].astype(bf), w_xs[...],
                preferred_element_type=jnp.float32)
    h = h + jnp.dot(xd[...].astype(bf), w_xd[...],
                    preferred_element_type=jnp.float32)
    h = h + jnp.dot(te[...][:, :D_T].astype(bf), w_te[...],
                    preferred_element_type=jnp.float32)
    h = h + di[...] * w_di[...]
    h = _silu(h + b1[...])
    h = jnp.dot(h, w2[...], preferred_element_type=jnp.float32) + b2[...]
    h = h + ea[...]
    m = jnp.mean(h, axis=-1, keepdims=True)
    v = jnp.mean((h - m) ** 2, axis=-1, keepdims=True)
    out_ref[...] = (h - m) * lax.rsqrt(v + 1e-5) * g[...] + bb[...]


_META_B = ((False, D_H, jnp.float32), (True, D_H, jnp.float32),
           (False, 2 * D_T, jnp.float32), (False, D_P, jnp.float32),
           (True, D_P, jnp.float32))
_META_F = ((False, D_H, jnp.float32), (True, D_H, jnp.float32))


@functools.lru_cache(maxsize=None)
def _gather_b(n_e):
    return _make_sc_gather(_META_B, n_e)


@functools.lru_cache(maxsize=None)
def _gather_f(n_e):
    return _make_sc_gather(_META_F, n_e)


def _row(v):
    return v.reshape(1, -1)


def kernel(x_feat, t, pos, edge_index, edge_attr, dist, batch, frame_emb,
           params):
    p = params
    src = edge_index[0].astype(jnp.int32)
    dst = edge_index[1].astype(jnp.int32)
    t2 = t.reshape(BS, 1)
    batch2 = batch.astype(jnp.int32).reshape(N, 1)
    posp = jnp.pad(pos, ((0, 0), (0, D_P - DIMS)))

    tfe = pl.pallas_call(
        _tfe_body,
        grid=(N // BLKN,),
        in_specs=[
            pl.BlockSpec((BS, 1), lambda i: (0, 0)),
            pl.BlockSpec((BLKN, 1), lambda i: (i, 0)),
            pl.BlockSpec((BLKN, D_T), lambda i: (i, 0)),
            pl.BlockSpec((D_T, H2), lambda i: (0, 0)),
            pl.BlockSpec((1, H2), lambda i: (0, 0)),
            pl.BlockSpec((H2, D_T), lambda i: (0, 0)),
            pl.BlockSpec((1, D_T), lambda i: (0, 0)),
        ],
        out_specs=pl.BlockSpec((BLKN, 2 * D_T), lambda i: (i, 0)),
        out_shape=jax.ShapeDtypeStruct((N, 2 * D_T), jnp.float32),
    )(t2, batch2, frame_emb, p['time_w1'], _row(p['time_b1']), p['time_w2'],
      _row(p['time_b2']))

    e0 = E_SPLITS[0]
    srcs = (src[:e0], src[e0:])
    dsts = (dst[:e0], dst[e0:])
    eas = (edge_attr[:e0], edge_attr[e0:])
    dis = (dist[:e0], dist[e0:])

    mw1 = p['msg_w1']
    pw2p = jnp.pad(p['pos_w2'], ((0, 0), (0, D_P - DIMS)))
    pb2p = _row(jnp.pad(p['pos_b2'], (0, D_P - DIMS)))
    wspec = lambda a: pl.BlockSpec(a.shape, lambda i: (0,) * a.ndim)
    bf = jnp.bfloat16
    msg_ws = [mw1[:D_H].astype(bf), mw1[D_H:2 * D_H].astype(bf),
              mw1[2 * D_H:2 * D_H + D_E],
              mw1[2 * D_H + D_E:2 * D_H + D_E + 1],
              mw1[2 * D_H + D_E + 1:].astype(bf),
              _row(p['msg_b1']), p['msg_w2'], _row(p['msg_b2']),
              p['pos_w1'], _row(p['pos_b1']), pw2p, pb2p]
    ew1 = p['edge_w1']
    edge_ws = [ew1[:D_H].astype(bf), ew1[D_H:2 * D_H].astype(bf),
               ew1[2 * D_H:2 * D_H + 1], ew1[2 * D_H + 1:].astype(bf),
               _row(p['edge_b1']), p['edge_w2'],
               _row(p['edge_b2']), _row(p['ln_e_g']), _row(p['ln_e_b'])]
    zeros32 = jnp.zeros((N, CW), jnp.float32)

    parts = []
    tfe_es = []
    for k in range(2):
        n_e = E_SPLITS[k]
        sx, dx, tfe_e, ps, pd = _gather_b(n_e)(srcs[k], dsts[k], x_feat,
                                               x_feat, tfe, posp, posp)
        tfe_es.append(tfe_e)
        espec = lambda w: pl.BlockSpec((BLKE, w), lambda i: (i, 0))
        comb = pl.pallas_call(
            _msg_body,
            grid=(n_e // BLKE,),
            in_specs=[espec(D_H), espec(D_H), espec(D_E), espec(1),
                      espec(2 * D_T), espec(D_P), espec(D_P)]
                     + [wspec(w) for w in msg_ws],
            out_specs=pl.BlockSpec((BLKE, CW), lambda i: (i, 0)),
            out_shape=jax.ShapeDtypeStruct((n_e, CW), jnp.float32),
        )(sx, dx, eas[k], dis[k], tfe_e, ps, pd, *msg_ws)
        part = _make_sc_scatter_add(n_e)(dsts[k], comb, zeros32)
        parts += [part[:N], part[N:]]

    nw1 = p['node_w1']
    node_ws = [nw1[:D_H], nw1[D_H:D_H + D_E], nw1[D_H + D_E:],
               _row(p['node_b1']), p['node_w2'], _row(p['node_b2']),
               _row(p['ln_n_g']), _row(p['ln_n_b'])]
    nspec = lambda w: pl.BlockSpec((BLKN, w), lambda i: (i, 0))
    x_new, pos_v_t = pl.pallas_call(
        _node_body,
        grid=(N // BLKN,),
        in_specs=[nspec(D_H)] + [nspec(CW)] * 4
                 + [pl.BlockSpec((BLKN, 2 * D_T), lambda i: (i, 0))]
                 + [wspec(w) for w in node_ws],
        out_specs=[nspec(D_H), nspec(DIMS)],
        out_shape=[jax.ShapeDtypeStruct((N, D_H), jnp.float32),
                   jax.ShapeDtypeStruct((N, DIMS), jnp.float32)],
    )(x_feat, *parts, tfe, *node_ws)

    edge_news = []
    for k in range(2):
        n_e = E_SPLITS[k]
        xs, xd = _gather_f(n_e)(srcs[k], dsts[k], x_new, x_new)
        espec = lambda w: pl.BlockSpec((BLKE, w), lambda i: (i, 0))
        edge_news.append(pl.pallas_call(
            _edge_body,
            grid=(n_e // BLKE,),
            in_specs=[espec(D_H), espec(D_H), espec(1),
                      pl.BlockSpec((BLKE, 2 * D_T), lambda i: (i, 0)),
                      espec(D_E)]
                     + [wspec(w) for w in edge_ws],
            out_specs=pl.BlockSpec((BLKE, D_E), lambda i: (i, 0)),
            out_shape=jax.ShapeDtypeStruct((n_e, D_E), jnp.float32),
        )(xs, xd, dis[k], tfe_es[k], eas[k], *edge_ws))
    edge_new = jnp.concatenate(edge_news, axis=0)

    return (pos_v_t, x_new, edge_new)
